# baseline (speedup 1.0000x reference)
import functools

import jax
import jax.numpy as jnp
from jax import lax
from jax.experimental import pallas as pl
from jax.experimental.pallas import tpu as pltpu
from jax.experimental.pallas import tpu_sc as plsc

_NC = 2
_NS = 16
_NW = _NC * _NS
_LANES = 16


def _make_gather(n, s, d, k, k_pad):
  nb = n // _NW

  mesh = plsc.VectorSubcoreMesh(core_axis_name="c", subcore_axis_name="s")

  @functools.partial(
      pl.kernel,
      out_type=jax.ShapeDtypeStruct((n, k, d), jnp.float32),
      mesh=mesh,
      scratch_types=[
          pltpu.VMEM((k_pad,), jnp.int32),
          pltpu.VMEM((2, nb, d), jnp.float32),
          pltpu.SemaphoreType.DMA,
      ],
  )
  def gather_kernel(x_hbm, idx_hbm, out_hbm, idx_v, buf, sem):
    wid = lax.axis_index("s") * _NC + lax.axis_index("c")
    b0 = wid * nb

    pltpu.sync_copy(idx_hbm, idx_v)
    vecs = [idx_v[pl.ds(_LANES * m, _LANES)] for m in range(k_pad // _LANES)]

    def read(i):
      j = vecs[i // _LANES][i % _LANES]
      return pltpu.make_async_copy(
          x_hbm.at[pl.ds(b0, nb), j], buf.at[i % 2], sem)

    read(0).start()
    for i in range(k):
      if i + 1 < k:
        read(i + 1).start()
      read(i).wait()
      pltpu.sync_copy(buf.at[i % 2], out_hbm.at[pl.ds(b0, nb), i])

  return gather_kernel


def kernel(x, index):
  n, s, d = x.shape
  k = index.shape[0]
  k_pad = -(-k // _LANES) * _LANES
  idx_p = jnp.pad(index, (0, k_pad - k))
  return _make_gather(n, s, d, k, k_pad)(x, idx_p)


# speedup vs baseline: 1.0060x; 1.0060x over previous
import functools

import jax
import jax.numpy as jnp
from jax import lax
from jax.experimental import pallas as pl
from jax.experimental.pallas import tpu as pltpu
from jax.experimental.pallas import tpu_sc as plsc

_NC = 2
_NS = 16
_NW = _NC * _NS
_LANES = 16
_NBUF = 7
_RAHEAD = 4


def _make_gather(n, s, d, k, k_pad):
  nb = n // _NW

  mesh = plsc.VectorSubcoreMesh(core_axis_name="c", subcore_axis_name="s")

  @functools.partial(
      pl.kernel,
      out_type=jax.ShapeDtypeStruct((n, k, d), jnp.float32),
      mesh=mesh,
      scratch_types=[
          pltpu.VMEM((k_pad,), jnp.int32),
          pltpu.VMEM((_NBUF, nb, d), jnp.float32),
          pltpu.SemaphoreType.DMA,
          pltpu.SemaphoreType.DMA,
      ],
  )
  def gather_kernel(x_hbm, idx_hbm, out_hbm, idx_v, buf, rsem, wsem):
    wid = lax.axis_index("s") * _NC + lax.axis_index("c")
    b0 = wid * nb

    pltpu.sync_copy(idx_hbm, idx_v)
    vecs = [idx_v[pl.ds(_LANES * m, _LANES)] for m in range(k_pad // _LANES)]

    def read(i):
      j = vecs[i // _LANES][i % _LANES]
      return pltpu.make_async_copy(
          x_hbm.at[pl.ds(b0, nb), j], buf.at[i % _NBUF], rsem)

    def write(i):
      return pltpu.make_async_copy(
          buf.at[i % _NBUF], out_hbm.at[pl.ds(b0, nb), i], wsem)

    for i in range(min(_RAHEAD, k)):
      read(i).start()
    for i in range(k):
      read(i).wait()
      write(i).start()
      if i >= _NBUF - _RAHEAD - 1:
        write(i - (_NBUF - _RAHEAD - 1)).wait()
      if i + _RAHEAD < k:
        read(i + _RAHEAD).start()
    for i in range(max(0, k - (_NBUF - _RAHEAD - 1)), k):
      write(i).wait()

  return gather_kernel


def kernel(x, index):
  n, s, d = x.shape
  k = index.shape[0]
  k_pad = -(-k // _LANES) * _LANES
  idx_p = jnp.pad(index, (0, k_pad - k))
  return _make_gather(n, s, d, k, k_pad)(x, idx_p)
